# baseline (speedup 1.0000x reference)
import functools

import jax
import jax.numpy as jnp
from jax import lax
from jax.experimental import pallas as pl
from jax.experimental.pallas import tpu as pltpu
from jax.experimental.pallas import tpu_sc as plsc

NUM_CODES = 1296
CODE_DIM = 256
K_PAD = 1408
TILE_N = 512


def _vq_body(x_ref, emb_ref, codes_ref):
    x = x_ref[...]
    emb = emb_ref[...]

    x3 = lax.dot_general(x, emb, (((1,), (1,)), ((), ())),
                         preferred_element_type=jnp.float32)
    x1 = jnp.sum(x * x, axis=1, keepdims=True)
    x2 = jnp.sum(emb * emb, axis=1)[None, :]

    kiota = lax.broadcasted_iota(jnp.int32, (TILE_N, K_PAD), 1)
    pad_mask = jnp.where(kiota >= NUM_CODES, jnp.float32(1e30), jnp.float32(0.0))
    d = x1 + x2 - 2.0 * x3 + pad_mask

    m = jnp.min(d, axis=1, keepdims=True)
    idx = jnp.min(jnp.where(d == m, kiota, jnp.int32(2**31 - 1)), axis=1)
    codes_ref[0, 0, :] = idx


def _argmin_codes(xf, embp):
    total = xf.shape[0]
    nb = total // TILE_N
    codes3d = pl.pallas_call(
        _vq_body,
        grid=(nb,),
        in_specs=[
            pl.BlockSpec((TILE_N, CODE_DIM), lambda i: (i, 0)),
            pl.BlockSpec((K_PAD, CODE_DIM), lambda i: (0, 0)),
        ],
        out_specs=pl.BlockSpec((1, 1, TILE_N), lambda i: (i, 0, 0)),
        out_shape=jax.ShapeDtypeStruct((nb, 1, TILE_N), jnp.int32),
    )(xf, embp)
    return codes3d.reshape(total)


def _sc_gather(table, idx_flat):
    info = plsc.get_sparse_core_info()
    nc, ns = info.num_cores, info.num_subcores
    nw = nc * ns
    total = idx_flat.shape[0]
    b_per_w = total // nw
    chunk = min(256, b_per_w)
    n_chunks = b_per_w // chunk
    mesh = plsc.VectorSubcoreMesh(core_axis_name="c", subcore_axis_name="s")

    @functools.partial(
        pl.kernel, mesh=mesh,
        out_type=jax.ShapeDtypeStruct((total, CODE_DIM), jnp.float32),
        scratch_types=[
            pltpu.VMEM((chunk,), jnp.int32),
            pltpu.VMEM((chunk, CODE_DIM), jnp.float32),
            pltpu.SemaphoreType.DMA,
        ],
    )
    def gather_k(table_hbm, idx_hbm, out_hbm, idx_v, rows_v, sem):
        wid = lax.axis_index("s") * nc + lax.axis_index("c")
        base = wid * b_per_w
        for c in range(n_chunks):
            off = base + c * chunk
            pltpu.sync_copy(idx_hbm.at[pl.ds(off, chunk)], idx_v)
            pltpu.async_copy(table_hbm.at[idx_v], rows_v, sem).wait()
            pltpu.sync_copy(rows_v, out_hbm.at[pl.ds(off, chunk)])

    return gather_k(table, idx_flat)


@jax.jit
def kernel(inputs, embeddings):
    b, m, n, d = inputs.shape
    total = b * m * n
    xf = inputs.reshape(total, d)
    embp = jnp.pad(embeddings, ((0, K_PAD - NUM_CODES), (0, 0)))

    codes_flat = _argmin_codes(xf, embp)
    vecs = _sc_gather(embeddings, codes_flat)

    return (codes_flat.reshape(b, m, n), vecs.reshape(b, m, n, d))


# speedup vs baseline: 1.0563x; 1.0563x over previous
import functools

import jax
import jax.numpy as jnp
from jax import lax
from jax.experimental import pallas as pl
from jax.experimental.pallas import tpu as pltpu
from jax.experimental.pallas import tpu_sc as plsc

NUM_CODES = 1296
CODE_DIM = 256
K_PAD = 1408
TILE_N = 512


def _vq_body(x_ref, embm2_ref, x2p_ref, codes_ref):
    x = x_ref[...]
    embm2 = embm2_ref[...]

    mm = lax.dot_general(x, embm2, (((1,), (1,)), ((), ())),
                         preferred_element_type=jnp.float32)
    x1 = jnp.sum(x * x, axis=1, keepdims=True)
    d = (x1 + x2p_ref[...]) + mm

    m = jnp.min(d, axis=1, keepdims=True)
    kf = lax.broadcasted_iota(jnp.int32, (TILE_N, K_PAD), 1).astype(jnp.float32)
    idxf = jnp.min(jnp.where(d == m, kf, jnp.float32(3e38)), axis=1)
    codes_ref[0, 0, :] = idxf.astype(jnp.int32)


def _argmin_codes(xf, embm2, x2p):
    total = xf.shape[0]
    nb = total // TILE_N
    codes3d = pl.pallas_call(
        _vq_body,
        grid=(nb,),
        in_specs=[
            pl.BlockSpec((TILE_N, CODE_DIM), lambda i: (i, 0)),
            pl.BlockSpec((K_PAD, CODE_DIM), lambda i: (0, 0)),
            pl.BlockSpec((1, K_PAD), lambda i: (0, 0)),
        ],
        out_specs=pl.BlockSpec((1, 1, TILE_N), lambda i: (i, 0, 0)),
        out_shape=jax.ShapeDtypeStruct((nb, 1, TILE_N), jnp.int32),
    )(xf, embm2, x2p)
    return codes3d.reshape(total)


def _sc_gather(table, idx_flat):
    info = plsc.get_sparse_core_info()
    nc, ns = info.num_cores, info.num_subcores
    nw = nc * ns
    total = idx_flat.shape[0]
    b_per_w = total // nw
    chunk = min(256, b_per_w)
    n_chunks = b_per_w // chunk
    mesh = plsc.VectorSubcoreMesh(core_axis_name="c", subcore_axis_name="s")

    @functools.partial(
        pl.kernel, mesh=mesh,
        out_type=jax.ShapeDtypeStruct((total, CODE_DIM), jnp.float32),
        scratch_types=[
            pltpu.VMEM((chunk,), jnp.int32),
            pltpu.VMEM((chunk, CODE_DIM), jnp.float32),
            pltpu.SemaphoreType.DMA,
        ],
    )
    def gather_k(table_hbm, idx_hbm, out_hbm, idx_v, rows_v, sem):
        wid = lax.axis_index("s") * nc + lax.axis_index("c")
        base = wid * b_per_w
        for c in range(n_chunks):
            off = base + c * chunk
            pltpu.sync_copy(idx_hbm.at[pl.ds(off, chunk)], idx_v)
            pltpu.async_copy(table_hbm.at[idx_v], rows_v, sem).wait()
            pltpu.sync_copy(rows_v, out_hbm.at[pl.ds(off, chunk)])

    return gather_k(table, idx_flat)


@jax.jit
def kernel(inputs, embeddings):
    b, m, n, d = inputs.shape
    total = b * m * n
    xf = inputs.reshape(total, d)
    embm2 = jnp.pad(-2.0 * embeddings, ((0, K_PAD - NUM_CODES), (0, 0)))
    x2 = jnp.sum(embeddings ** 2, axis=-1)
    x2p = jnp.pad(x2, (0, K_PAD - NUM_CODES),
                  constant_values=jnp.float32(1e30))[None, :]

    codes_flat = _argmin_codes(xf, embm2, x2p)
    vecs = _sc_gather(embeddings, codes_flat)

    return (codes_flat.reshape(b, m, n), vecs.reshape(b, m, n, d))
